# baseline (speedup 1.0000x reference)
import functools

import jax
import jax.numpy as jnp
from jax import lax
from jax.experimental import pallas as pl
from jax.experimental.pallas import tpu as pltpu
from jax.experimental.pallas import tpu_sc as plsc

VOCAB = 1000000
EMBED = 64
BATCH = 4096
SEQ = 200

NUM_IDX = BATCH * SEQ
NUM_WORKERS = 32
PER_WORKER = NUM_IDX // NUM_WORKERS
GATHER_WINDOW = 512
NUM_CHUNKS = PER_WORKER // GATHER_WINDOW


def _sc_gather_pairs(table_pairs, pair_idx):
    mesh = plsc.VectorSubcoreMesh(core_axis_name="c", subcore_axis_name="s")

    @functools.partial(
        pl.kernel,
        mesh=mesh,
        out_type=jax.ShapeDtypeStruct((NUM_IDX, 2 * EMBED), jnp.float32),
        scratch_types=[
            pltpu.VMEM((GATHER_WINDOW,), jnp.int32),
            pltpu.VMEM((GATHER_WINDOW, 2 * EMBED), jnp.float32),
            pltpu.SemaphoreType.DMA,
        ],
    )
    def gather_kernel(table_hbm, idx_hbm, out_hbm, idx_v, rows_v, sem):
        wid = lax.axis_index("s") * 2 + lax.axis_index("c")

        @pl.loop(0, NUM_CHUNKS)
        def _(c):
            base = wid * PER_WORKER + c * GATHER_WINDOW
            pltpu.sync_copy(idx_hbm.at[pl.ds(base, GATHER_WINDOW)], idx_v)
            pltpu.async_copy(table_hbm.at[idx_v], rows_v, sem).wait()
            pltpu.sync_copy(rows_v, out_hbm.at[pl.ds(base, GATHER_WINDOW)])

    return gather_kernel(table_pairs, pair_idx)


def _tc_select_transpose_body(x_ref, par_ref, o_ref):
    xt = jnp.transpose(x_ref[...], (0, 2, 1))
    par = par_ref[...]
    o_ref[...] = jnp.where(par == 0, xt[:, :EMBED, :], xt[:, EMBED:, :])


def _tc_select_transpose(gathered, parity):
    bb = 16
    return pl.pallas_call(
        _tc_select_transpose_body,
        grid=(BATCH // bb,),
        in_specs=[
            pl.BlockSpec((bb, SEQ, 2 * EMBED), lambda i: (i, 0, 0)),
            pl.BlockSpec((bb, 1, SEQ), lambda i: (i, 0, 0)),
        ],
        out_specs=pl.BlockSpec((bb, EMBED, SEQ), lambda i: (i, 0, 0)),
        out_shape=jax.ShapeDtypeStruct((BATCH, EMBED, SEQ), jnp.float32),
    )(gathered, parity)


def kernel(inputs, table):
    idx = inputs.astype(jnp.int32)
    table_pairs = table.reshape(VOCAB // 2, 2 * EMBED)
    pair_idx = lax.shift_right_logical(idx, 1).reshape(NUM_IDX)
    parity = lax.bitwise_and(idx, 1).reshape(BATCH, 1, SEQ)
    gathered = _sc_gather_pairs(table_pairs, pair_idx)
    return _tc_select_transpose(
        gathered.reshape(BATCH, SEQ, 2 * EMBED), parity
    )


# speedup vs baseline: 1.4503x; 1.4503x over previous
import functools

import jax
import jax.numpy as jnp
from jax import lax
from jax.experimental import pallas as pl
from jax.experimental.pallas import tpu as pltpu
from jax.experimental.pallas import tpu_sc as plsc

VOCAB = 1000000
EMBED = 64
BATCH = 4096
SEQ = 200
LANES = 128

NUM_IDX = BATCH * SEQ
NUM_WORKERS = 32
PER_WORKER = NUM_IDX // NUM_WORKERS
GATHER_WINDOW = 512
NUM_CHUNKS = PER_WORKER // GATHER_WINDOW

RELAYOUT_VC = 2048


def _relayout_body(x_ref, o_ref):
    x = x_ref[...]
    o_ref[...] = jnp.concatenate(
        [jnp.transpose(x, (1, 0)), jnp.zeros((RELAYOUT_VC, EMBED), x.dtype)],
        axis=1,
    )


def _tc_relayout(tbl_t):
    grid = pl.cdiv(VOCAB, RELAYOUT_VC)
    return pl.pallas_call(
        _relayout_body,
        grid=(grid,),
        in_specs=[pl.BlockSpec((EMBED, RELAYOUT_VC), lambda i: (0, i))],
        out_specs=pl.BlockSpec((RELAYOUT_VC, LANES), lambda i: (i, 0)),
        out_shape=jax.ShapeDtypeStruct((VOCAB, LANES), jnp.float32),
    )(tbl_t)


def _sc_gather(table128, idx_flat):
    mesh = plsc.VectorSubcoreMesh(core_axis_name="c", subcore_axis_name="s")

    @functools.partial(
        pl.kernel,
        mesh=mesh,
        out_type=jax.ShapeDtypeStruct((NUM_IDX, LANES), jnp.float32),
        scratch_types=[
            pltpu.VMEM((GATHER_WINDOW,), jnp.int32),
            pltpu.VMEM((GATHER_WINDOW, LANES), jnp.float32),
            pltpu.SemaphoreType.DMA,
        ],
    )
    def gather_kernel(table_hbm, idx_hbm, out_hbm, idx_v, rows_v, sem):
        wid = lax.axis_index("s") * 2 + lax.axis_index("c")

        @pl.loop(0, NUM_CHUNKS)
        def _(c):
            base = wid * PER_WORKER + c * GATHER_WINDOW
            pltpu.sync_copy(idx_hbm.at[pl.ds(base, GATHER_WINDOW)], idx_v)
            pltpu.async_copy(table_hbm.at[idx_v], rows_v, sem).wait()
            pltpu.sync_copy(rows_v, out_hbm.at[pl.ds(base, GATHER_WINDOW)])

    return gather_kernel(table128, idx_flat)


OT_SS = 8
OT_BB = 1024


def _out_transpose_body(x_ref, o_ref):
    x = x_ref[...]
    o_ref[...] = jnp.transpose(x[:, :, :EMBED], (2, 0, 1))


def _tc_out_transpose(gathered_3d):
    return pl.pallas_call(
        _out_transpose_body,
        grid=(SEQ // OT_SS, BATCH // OT_BB),
        in_specs=[pl.BlockSpec((OT_SS, OT_BB, LANES), lambda i, j: (i, j, 0))],
        out_specs=pl.BlockSpec((EMBED, OT_SS, OT_BB), lambda i, j: (0, i, j)),
        out_shape=jax.ShapeDtypeStruct((EMBED, SEQ, BATCH), jnp.float32),
    )(gathered_3d)


def kernel(inputs, table):
    idx_flat = jnp.transpose(inputs).reshape(NUM_IDX).astype(jnp.int32)
    tbl_t = jnp.transpose(table)
    table128 = _tc_relayout(tbl_t)
    gathered = _sc_gather(table128, idx_flat)
    out_esb = _tc_out_transpose(gathered.reshape(SEQ, BATCH, LANES))
    return jnp.transpose(out_esb, (2, 0, 1))


# speedup vs baseline: 1.9025x; 1.3117x over previous
import functools

import jax
import jax.numpy as jnp
from jax import lax
from jax.experimental import pallas as pl
from jax.experimental.pallas import tpu as pltpu
from jax.experimental.pallas import tpu_sc as plsc

VOCAB = 1000000
EMBED = 64
BATCH = 4096
SEQ = 200
LANES = 128

NUM_IDX = BATCH * SEQ
NUM_WORKERS = 32
GATHER_WINDOW = 256

RELAYOUT_PC = 4096
PAIR_H = 123 * RELAYOUT_PC
NUM_PAIRS = PAIR_H
HI_BLOCKS = PAIR_H // RELAYOUT_PC
LAST_VBLOCK = pl.cdiv(VOCAB, RELAYOUT_PC) - 1


def _relayout_body(lo_ref, hi_ref, o_ref):
    lo = jnp.transpose(lo_ref[...], (1, 0))
    hi = jnp.transpose(hi_ref[...], (1, 0))
    o_ref[...] = jnp.concatenate([lo, hi], axis=1)


def _tc_relayout(tbl_t):
    return pl.pallas_call(
        _relayout_body,
        grid=(HI_BLOCKS,),
        in_specs=[
            pl.BlockSpec((EMBED, RELAYOUT_PC), lambda i: (0, i)),
            pl.BlockSpec(
                (EMBED, RELAYOUT_PC),
                lambda i: (0, jnp.minimum(HI_BLOCKS + i, LAST_VBLOCK)),
            ),
        ],
        out_specs=pl.BlockSpec((RELAYOUT_PC, LANES), lambda i: (i, 0)),
        out_shape=jax.ShapeDtypeStruct((NUM_PAIRS, LANES), jnp.float32),
    )(tbl_t, tbl_t)


K_CHUNKS = 5
CHUNK_S = SEQ // K_CHUNKS
CHUNK_N = CHUNK_S * BATCH
CH_PER_WORKER = CHUNK_N // NUM_WORKERS
CH_NUM_WIN = CH_PER_WORKER // GATHER_WINDOW


def _sc_gather_chunk(table128, idx_chunk):
    mesh = plsc.VectorSubcoreMesh(core_axis_name="c", subcore_axis_name="s")
    w_sz = GATHER_WINDOW

    @functools.partial(
        pl.kernel,
        mesh=mesh,
        out_type=jax.ShapeDtypeStruct((CHUNK_N, LANES), jnp.float32),
        scratch_types=[
            pltpu.VMEM((CH_PER_WORKER,), jnp.int32),
            pltpu.VMEM((w_sz, LANES), jnp.float32),
            pltpu.VMEM((w_sz, LANES), jnp.float32),
            pltpu.SemaphoreType.DMA,
            pltpu.SemaphoreType.DMA,
        ],
    )
    def gather_kernel(table_hbm, idx_hbm, out_hbm, idx_v, rows0, rows1,
                      sem0, sem1):
        wid = lax.axis_index("s") * 2 + lax.axis_index("c")
        base = wid * CH_PER_WORKER
        pltpu.sync_copy(idx_hbm.at[pl.ds(base, CH_PER_WORKER)], idx_v)

        def start(w, rows, sem):
            pltpu.async_copy(
                table_hbm.at[idx_v.at[pl.ds(w * w_sz, w_sz)]], rows, sem
            )

        def drain(w, rows, sem):
            pltpu.make_async_copy(
                table_hbm.at[idx_v.at[pl.ds(0, w_sz)]], rows, sem
            ).wait()
            pltpu.sync_copy(rows, out_hbm.at[pl.ds(base + w * w_sz, w_sz)])

        start(0, rows0, sem0)
        start(1, rows1, sem1)

        @pl.loop(0, CH_NUM_WIN - 2, step=2)
        def _(w):
            drain(w, rows0, sem0)
            start(w + 2, rows0, sem0)
            drain(w + 1, rows1, sem1)
            start(w + 3, rows1, sem1)

        drain(CH_NUM_WIN - 2, rows0, sem0)
        drain(CH_NUM_WIN - 1, rows1, sem1)

    return gather_kernel(table128, idx_chunk)


OT_SS = 8
OT_BB = 1024
OT_GRID_S = CHUNK_S // OT_SS


def _ot_select(x, idx_blk):
    t = jnp.transpose(x, (2, 0, 1))
    return jnp.where(idx_blk < PAIR_H, t[:EMBED], t[EMBED:])


def _ot_first_body(x_ref, i_ref, o_ref):
    o_ref[...] = _ot_select(x_ref[...], i_ref[...])


def _ot_chunk_body(x_ref, i_ref, carry_ref, o_ref):
    del carry_ref
    o_ref[...] = _ot_select(x_ref[...], i_ref[...])


def _tc_ot_chunk(gathered_3d, idx3, carry, k):
    out_shape = jax.ShapeDtypeStruct((EMBED, SEQ, BATCH), jnp.float32)
    out_spec = pl.BlockSpec(
        (EMBED, OT_SS, OT_BB),
        lambda i, j, k=k: (0, k * OT_GRID_S + i, j),
    )
    in_spec = pl.BlockSpec((OT_SS, OT_BB, LANES), lambda i, j: (i, j, 0))
    idx_spec = pl.BlockSpec(
        (1, OT_SS, OT_BB),
        lambda i, j, k=k: (0, k * OT_GRID_S + i, j),
    )
    grid = (OT_GRID_S, BATCH // OT_BB)
    if carry is None:
        return pl.pallas_call(
            _ot_first_body,
            grid=grid,
            in_specs=[in_spec, idx_spec],
            out_specs=out_spec,
            out_shape=out_shape,
        )(gathered_3d, idx3)
    return pl.pallas_call(
        _ot_chunk_body,
        grid=grid,
        in_specs=[
            in_spec,
            idx_spec,
            pl.BlockSpec(memory_space=pltpu.MemorySpace.HBM),
        ],
        out_specs=out_spec,
        out_shape=out_shape,
        input_output_aliases={2: 0},
    )(gathered_3d, idx3, carry)


def kernel(inputs, table):
    idx_t = jnp.transpose(inputs).astype(jnp.int32)
    idx3 = idx_t.reshape(1, SEQ, BATCH)
    pair_idx = jnp.where(idx_t < PAIR_H, idx_t, idx_t - PAIR_H)
    pair_chunks = pair_idx.reshape(K_CHUNKS, CHUNK_N)
    tbl_t = jnp.transpose(table)
    table128 = _tc_relayout(tbl_t)
    carry = None
    for k in range(K_CHUNKS):
        g = _sc_gather_chunk(table128, pair_chunks[k])
        carry = _tc_ot_chunk(g.reshape(CHUNK_S, BATCH, LANES), idx3, carry, k)
    return jnp.transpose(carry, (2, 0, 1))


# speedup vs baseline: 2.0148x; 1.0591x over previous
import functools

import jax
import jax.numpy as jnp
from jax import lax
from jax.experimental import pallas as pl
from jax.experimental.pallas import tpu as pltpu
from jax.experimental.pallas import tpu_sc as plsc

VOCAB = 1000000
EMBED = 64
BATCH = 4096
SEQ = 200
LANES = 128

NUM_IDX = BATCH * SEQ
NUM_WORKERS = 32
GATHER_WINDOW = 256

RELAYOUT_PC = 16384
PAIR_H = 31 * RELAYOUT_PC
NUM_PAIRS = PAIR_H
HI_BLOCKS = PAIR_H // RELAYOUT_PC
LAST_VBLOCK = pl.cdiv(VOCAB, RELAYOUT_PC) - 1


def _relayout_body(lo_ref, hi_ref, o_ref):
    lo = jnp.transpose(lo_ref[...], (1, 0))
    hi = jnp.transpose(hi_ref[...], (1, 0))
    o_ref[...] = jnp.concatenate([lo, hi], axis=1)


def _tc_relayout(tbl_t):
    return pl.pallas_call(
        _relayout_body,
        grid=(HI_BLOCKS,),
        in_specs=[
            pl.BlockSpec((EMBED, RELAYOUT_PC), lambda i: (0, i)),
            pl.BlockSpec(
                (EMBED, RELAYOUT_PC),
                lambda i: (0, jnp.minimum(HI_BLOCKS + i, LAST_VBLOCK)),
            ),
        ],
        out_specs=pl.BlockSpec((RELAYOUT_PC, LANES), lambda i: (i, 0)),
        out_shape=jax.ShapeDtypeStruct((NUM_PAIRS, LANES), jnp.float32),
    )(tbl_t, tbl_t)


K_CHUNKS = 5
CHUNK_S = SEQ // K_CHUNKS
CHUNK_N = CHUNK_S * BATCH
CH_PER_WORKER = CHUNK_N // NUM_WORKERS
CH_NUM_WIN = CH_PER_WORKER // GATHER_WINDOW


def _sc_gather_chunk(table128, idx_chunk):
    mesh = plsc.VectorSubcoreMesh(core_axis_name="c", subcore_axis_name="s")
    w_sz = GATHER_WINDOW

    @functools.partial(
        pl.kernel,
        mesh=mesh,
        out_type=jax.ShapeDtypeStruct((CHUNK_N, LANES), jnp.float32),
        scratch_types=[
            pltpu.VMEM((CH_PER_WORKER,), jnp.int32),
            pltpu.VMEM((w_sz, LANES), jnp.float32),
            pltpu.VMEM((w_sz, LANES), jnp.float32),
            pltpu.SemaphoreType.DMA,
            pltpu.SemaphoreType.DMA,
        ],
    )
    def gather_kernel(table_hbm, idx_hbm, out_hbm, idx_v, rows0, rows1,
                      sem0, sem1):
        wid = lax.axis_index("s") * 2 + lax.axis_index("c")
        base = wid * CH_PER_WORKER
        pltpu.sync_copy(idx_hbm.at[pl.ds(base, CH_PER_WORKER)], idx_v)

        def start(w, rows, sem):
            pltpu.async_copy(
                table_hbm.at[idx_v.at[pl.ds(w * w_sz, w_sz)]], rows, sem
            )

        def drain(w, rows, sem):
            pltpu.make_async_copy(
                table_hbm.at[idx_v.at[pl.ds(0, w_sz)]], rows, sem
            ).wait()
            pltpu.sync_copy(rows, out_hbm.at[pl.ds(base + w * w_sz, w_sz)])

        start(0, rows0, sem0)
        start(1, rows1, sem1)

        @pl.loop(0, CH_NUM_WIN - 2, step=2)
        def _(w):
            drain(w, rows0, sem0)
            start(w + 2, rows0, sem0)
            drain(w + 1, rows1, sem1)
            start(w + 3, rows1, sem1)

        drain(CH_NUM_WIN - 2, rows0, sem0)
        drain(CH_NUM_WIN - 1, rows1, sem1)

    return gather_kernel(table128, idx_chunk)


OT_SS = 8
OT_BB = 1024
OT_GRID_S = CHUNK_S // OT_SS


def _ot_select(x, idx_blk):
    t = jnp.transpose(x, (2, 0, 1))
    return jnp.where(idx_blk < PAIR_H, t[:EMBED], t[EMBED:])


def _ot_first_body(x_ref, i_ref, o_ref):
    o_ref[...] = _ot_select(x_ref[...], i_ref[...])


def _ot_chunk_body(x_ref, i_ref, carry_ref, o_ref):
    del carry_ref
    o_ref[...] = _ot_select(x_ref[...], i_ref[...])


def _tc_ot_chunk(gathered_3d, idx3, carry, k):
    out_shape = jax.ShapeDtypeStruct((EMBED, SEQ, BATCH), jnp.float32)
    out_spec = pl.BlockSpec(
        (EMBED, OT_SS, OT_BB),
        lambda i, j, k=k: (0, k * OT_GRID_S + i, j),
    )
    in_spec = pl.BlockSpec((OT_SS, OT_BB, LANES), lambda i, j: (i, j, 0))
    idx_spec = pl.BlockSpec(
        (1, OT_SS, OT_BB),
        lambda i, j, k=k: (0, k * OT_GRID_S + i, j),
    )
    grid = (OT_GRID_S, BATCH // OT_BB)
    if carry is None:
        return pl.pallas_call(
            _ot_first_body,
            grid=grid,
            in_specs=[in_spec, idx_spec],
            out_specs=out_spec,
            out_shape=out_shape,
        )(gathered_3d, idx3)
    return pl.pallas_call(
        _ot_chunk_body,
        grid=grid,
        in_specs=[
            in_spec,
            idx_spec,
            pl.BlockSpec(memory_space=pltpu.MemorySpace.HBM),
        ],
        out_specs=out_spec,
        out_shape=out_shape,
        input_output_aliases={2: 0},
    )(gathered_3d, idx3, carry)


def kernel(inputs, table):
    idx_t = jnp.transpose(inputs).astype(jnp.int32)
    idx3 = idx_t.reshape(1, SEQ, BATCH)
    pair_idx = jnp.where(idx_t < PAIR_H, idx_t, idx_t - PAIR_H)
    pair_chunks = pair_idx.reshape(K_CHUNKS, CHUNK_N)
    tbl_t = jnp.transpose(table)
    table128 = _tc_relayout(tbl_t)
    carry = None
    for k in range(K_CHUNKS):
        g = _sc_gather_chunk(table128, pair_chunks[k])
        carry = _tc_ot_chunk(g.reshape(CHUNK_S, BATCH, LANES), idx3, carry, k)
    return jnp.transpose(carry, (2, 0, 1))
